# Initial kernel scaffold; baseline (speedup 1.0000x reference)
import functools

import jax
import jax.numpy as jnp
from jax import lax
from jax.experimental import pallas as pl
from jax.experimental.pallas import tpu as pltpu
from jax.experimental.pallas import tpu_sc as plsc

NC = 2
NS = 16
NW = NC * NS
EK = 80
DEGW = 16
ZB = 125


def _vsc_mesh():
    return plsc.VectorSubcoreMesh(core_axis_name="c", subcore_axis_name="s")


def _sc_deg(n, e):
    rows = e // EK
    rpt = rows // NW
    npt = n // NS

    @functools.partial(
        pl.kernel,
        mesh=_vsc_mesh(),
        out_type=jax.ShapeDtypeStruct((NC, n, DEGW), jnp.float32),
        scratch_types=[
            pltpu.VMEM((rpt, EK), jnp.int32),
            pltpu.VMEM((EK, DEGW), jnp.float32),
            pltpu.VMEM((ZB, DEGW), jnp.float32),
            pltpu.VMEM_SHARED((n, DEGW), jnp.float32),
        ],
    )
    def k(dst_hbm, out_hbm, idx_v, ones_v, zb_v, acc_sh):
        c = lax.axis_index("c")
        s = lax.axis_index("s")
        w = c * NS + s

        def fill(i, _):
            ones_v[i, :] = jnp.ones((DEGW,), jnp.float32)
            return 0

        lax.fori_loop(0, EK, fill, 0)

        def fillz(i, _):
            zb_v[i, :] = jnp.zeros((DEGW,), jnp.float32)
            return 0

        lax.fori_loop(0, ZB, fillz, 0)

        def zloop(b, _):
            pltpu.sync_copy(zb_v, acc_sh.at[pl.ds(s * npt + b * ZB, ZB)])
            return 0

        lax.fori_loop(0, npt // ZB, zloop, 0)
        plsc.subcore_barrier()

        pltpu.sync_copy(dst_hbm.at[pl.ds(w * rpt, rpt)], idx_v)

        def body(j, _):
            pltpu.sync_copy(ones_v, acc_sh.at[idx_v.at[j]], add=True)
            return 0

        lax.fori_loop(0, rpt, body, 0)
        plsc.subcore_barrier()

        def wloop(b, _):
            r0 = s * npt + b * ZB
            pltpu.sync_copy(acc_sh.at[pl.ds(r0, ZB)], zb_v)
            pltpu.sync_copy(zb_v, out_hbm.at[c, pl.ds(r0, ZB)])
            return 0

        lax.fori_loop(0, npt // ZB, wloop, 0)

    return k


def _sc_agg(n, e, d):
    rows = e // EK
    rpt = rows // NW
    npt = n // NS

    @functools.partial(
        pl.kernel,
        mesh=_vsc_mesh(),
        out_type=jax.ShapeDtypeStruct((NC, n, d), jnp.float32),
        scratch_types=[
            pltpu.VMEM((rpt, EK), jnp.int32),
            pltpu.VMEM((rpt, EK), jnp.int32),
            pltpu.VMEM((EK, d), jnp.float32),
            pltpu.VMEM((ZB, d), jnp.float32),
            pltpu.VMEM_SHARED((n, d), jnp.float32),
            pltpu.SemaphoreType.DMA,
        ],
    )
    def k(hs_hbm, src_hbm, dst_hbm, out_hbm, sidx, didx, rb, zb, acc_sh, sem):
        c = lax.axis_index("c")
        s = lax.axis_index("s")
        w = c * NS + s

        def fillz(i, _):
            zb[i // 8, pl.ds((i % 8) * 16, 16)] = jnp.zeros((16,), jnp.float32)
            return 0

        lax.fori_loop(0, ZB * d // 16, fillz, 0)

        def zloop(b, _):
            pltpu.sync_copy(zb, acc_sh.at[pl.ds(s * npt + b * ZB, ZB)])
            return 0

        lax.fori_loop(0, npt // ZB, zloop, 0)
        plsc.subcore_barrier()

        pltpu.sync_copy(src_hbm.at[pl.ds(w * rpt, rpt)], sidx)
        pltpu.sync_copy(dst_hbm.at[pl.ds(w * rpt, rpt)], didx)

        def body(j, _):
            pltpu.async_copy(hs_hbm.at[sidx.at[j]], rb, sem).wait()
            pltpu.sync_copy(rb, acc_sh.at[didx.at[j]], add=True)
            return 0

        lax.fori_loop(0, rpt, body, 0)
        plsc.subcore_barrier()

        def wloop(b, _):
            r0 = s * npt + b * ZB
            pltpu.sync_copy(acc_sh.at[pl.ds(r0, ZB)], zb)
            pltpu.sync_copy(zb, out_hbm.at[c, pl.ds(r0, ZB)])
            return 0

        lax.fori_loop(0, npt // ZB, wloop, 0)

    return k


def _dinv_from(degp_ref):
    deg = 1.0 + degp_ref[0, :, 0] + degp_ref[1, :, 0]
    return lax.rsqrt(deg)


def _tc_first(n, d, r):
    def body(x_ref, w_ref, degp_ref, hs_ref):
        dinv = _dinv_from(degp_ref)
        h = jnp.dot(x_ref[...], w_ref[...], preferred_element_type=jnp.float32)
        hs_ref[...] = h * dinv[:, None]

    return pl.pallas_call(
        body,
        grid=(n // r,),
        in_specs=[
            pl.BlockSpec((r, d), lambda i: (i, 0)),
            pl.BlockSpec((d, d), lambda i: (0, 0)),
            pl.BlockSpec((2, r, DEGW), lambda i: (0, i, 0)),
        ],
        out_specs=pl.BlockSpec((r, d), lambda i: (i, 0)),
        out_shape=jax.ShapeDtypeStruct((n, d), jnp.float32),
    )


def _tc_mid(n, d, r, relu):
    def body(aggp_ref, hsp_ref, b_ref, degp_ref, w_ref, out_ref):
        dinv = _dinv_from(degp_ref)
        t = (aggp_ref[0] + aggp_ref[1] + hsp_ref[...]) * dinv[:, None] + b_ref[...]
        if relu:
            t = jnp.maximum(t, 0.0)
        h = jnp.dot(t, w_ref[...], preferred_element_type=jnp.float32)
        out_ref[...] = h * dinv[:, None]

    return pl.pallas_call(
        body,
        grid=(n // r,),
        in_specs=[
            pl.BlockSpec((2, r, d), lambda i: (0, i, 0)),
            pl.BlockSpec((r, d), lambda i: (i, 0)),
            pl.BlockSpec((1, d), lambda i: (0, 0)),
            pl.BlockSpec((2, r, DEGW), lambda i: (0, i, 0)),
            pl.BlockSpec((d, d), lambda i: (0, 0)),
        ],
        out_specs=pl.BlockSpec((r, d), lambda i: (i, 0)),
        out_shape=jax.ShapeDtypeStruct((n, d), jnp.float32),
    )


def _tc_last(n, d, r):
    def body(aggp_ref, hsp_ref, degp_ref, out_ref):
        dinv = _dinv_from(degp_ref)
        out_ref[...] = (aggp_ref[0] + aggp_ref[1] + hsp_ref[...]) * dinv[:, None]

    return pl.pallas_call(
        body,
        grid=(n // r,),
        in_specs=[
            pl.BlockSpec((2, r, d), lambda i: (0, i, 0)),
            pl.BlockSpec((r, d), lambda i: (i, 0)),
            pl.BlockSpec((2, r, DEGW), lambda i: (0, i, 0)),
        ],
        out_specs=pl.BlockSpec((r, d), lambda i: (i, 0)),
        out_shape=jax.ShapeDtypeStruct((n, d), jnp.float32),
    )


def kernel(x, edge_index, W_in, b_in, W_h0, b_h0, W_h1, b_h1, W_out):
    n, d = x.shape
    e = edge_index.shape[1]
    r = 1000

    src2d = edge_index[0].reshape(e // EK, EK)
    dst2d = edge_index[1].reshape(e // EK, EK)

    degp = _sc_deg(n, e)(dst2d)
    agg = _sc_agg(n, e, d)
    first = _tc_first(n, d, r)
    mid_nr = _tc_mid(n, d, r, relu=False)
    mid_re = _tc_mid(n, d, r, relu=True)
    last = _tc_last(n, d, r)

    b2_in = b_in.reshape(1, d)
    b2_h0 = b_h0.reshape(1, d)
    b2_h1 = b_h1.reshape(1, d)

    hs0 = first(x, W_in, degp)
    a0 = agg(hs0, src2d, dst2d)
    hs1 = mid_nr(a0, hs0, b2_in, degp, W_h0)
    a1 = agg(hs1, src2d, dst2d)
    hs2 = mid_re(a1, hs1, b2_h0, degp, W_h1)
    a2 = agg(hs2, src2d, dst2d)
    hs3 = mid_re(a2, hs2, b2_h1, degp, W_out)
    a3 = agg(hs3, src2d, dst2d)
    return last(a3, hs3, degp)


# speedup vs baseline: 6.7343x; 6.7343x over previous
import functools

import jax
import jax.numpy as jnp
from jax import lax
from jax.experimental import pallas as pl
from jax.experimental.pallas import tpu as pltpu
from jax.experimental.pallas import tpu_sc as plsc

NC = 2
NS = 16
NW = NC * NS
EK = 128
DEGW = 16
ZB = 128


def _vsc_mesh():
    return plsc.VectorSubcoreMesh(core_axis_name="c", subcore_axis_name="s")


def _pad_up(v, m):
    return ((v + m - 1) // m) * m


def _sc_deg(np_, ep):
    rows = ep // EK
    rpt = rows // NW
    npt = np_ // NS

    @functools.partial(
        pl.kernel,
        mesh=_vsc_mesh(),
        out_type=jax.ShapeDtypeStruct((NC, np_, DEGW), jnp.float32),
        scratch_types=[
            pltpu.VMEM((rpt, EK), jnp.int32),
            pltpu.VMEM((EK, DEGW), jnp.float32),
            pltpu.VMEM((ZB, DEGW), jnp.float32),
            pltpu.VMEM_SHARED((np_, DEGW), jnp.float32),
        ],
    )
    def k(dst_hbm, out_hbm, idx_v, ones_v, zb_v, acc_sh):
        c = lax.axis_index("c")
        s = lax.axis_index("s")
        w = c * NS + s

        def fill(i, _):
            ones_v[i, :] = jnp.ones((DEGW,), jnp.float32)
            return 0

        lax.fori_loop(0, EK, fill, 0)

        def fillz(i, _):
            zb_v[i, :] = jnp.zeros((DEGW,), jnp.float32)
            return 0

        lax.fori_loop(0, ZB, fillz, 0)

        def zloop(b, _):
            pltpu.sync_copy(zb_v, acc_sh.at[pl.ds(s * npt + b * ZB, ZB)])
            return 0

        lax.fori_loop(0, npt // ZB, zloop, 0)
        plsc.subcore_barrier()

        pltpu.sync_copy(dst_hbm.at[pl.ds(w * rpt, rpt)], idx_v)

        def body(j, _):
            pltpu.sync_copy(ones_v, acc_sh.at[idx_v.at[j]], add=True)
            return 0

        lax.fori_loop(0, rpt, body, 0)
        plsc.subcore_barrier()

        def wloop(b, _):
            r0 = s * npt + b * ZB
            pltpu.sync_copy(acc_sh.at[pl.ds(r0, ZB)], zb_v)
            pltpu.sync_copy(zb_v, out_hbm.at[c, pl.ds(r0, ZB)])
            return 0

        lax.fori_loop(0, npt // ZB, wloop, 0)

    return k


def _sc_agg(n, np_, ep, d):
    rows = ep // EK
    rpt = rows // NW
    npt = np_ // NS

    @functools.partial(
        pl.kernel,
        mesh=_vsc_mesh(),
        out_type=jax.ShapeDtypeStruct((NC, np_, d), jnp.float32),
        scratch_types=[
            pltpu.VMEM((rpt, EK), jnp.int32),
            pltpu.VMEM((rpt, EK), jnp.int32),
            pltpu.VMEM((EK, d), jnp.float32),
            pltpu.VMEM_SHARED((np_, d), jnp.float32),
            pltpu.SemaphoreType.DMA,
        ],
    )
    def k(hs_hbm, src_hbm, dst_hbm, out_hbm, sidx, didx, rb, acc_sh, sem):
        c = lax.axis_index("c")
        s = lax.axis_index("s")
        w = c * NS + s

        def fillz(i, _):
            rb[i // 8, pl.ds((i % 8) * 16, 16)] = jnp.zeros((16,), jnp.float32)
            return 0

        lax.fori_loop(0, EK * d // 16, fillz, 0)

        def zloop(b, _):
            pltpu.sync_copy(rb, acc_sh.at[pl.ds(s * npt + b * ZB, ZB)])
            return 0

        lax.fori_loop(0, npt // ZB, zloop, 0)
        plsc.subcore_barrier()

        pltpu.sync_copy(src_hbm.at[pl.ds(w * rpt, rpt)], sidx)
        pltpu.sync_copy(dst_hbm.at[pl.ds(w * rpt, rpt)], didx)

        def body(j, _):
            pltpu.async_copy(hs_hbm.at[sidx.at[j]], rb, sem).wait()
            pltpu.sync_copy(rb, acc_sh.at[didx.at[j]], add=True)
            return 0

        lax.fori_loop(0, rpt, body, 0)
        plsc.subcore_barrier()

        def wloop(b, _):
            r0 = s * npt + b * ZB
            pltpu.sync_copy(acc_sh.at[pl.ds(r0, ZB)], rb)
            pltpu.sync_copy(rb, out_hbm.at[c, pl.ds(r0, ZB)])
            return 0

        lax.fori_loop(0, npt // ZB, wloop, 0)

    return k


def _dinv_from(degp_ref):
    deg = 1.0 + degp_ref[0, :, 0] + degp_ref[1, :, 0]
    return lax.rsqrt(deg)


def _tc_first(n, d, r):
    def body(x_ref, w_ref, degp_ref, hs_ref):
        dinv = _dinv_from(degp_ref)
        h = jnp.dot(x_ref[...], w_ref[...], preferred_element_type=jnp.float32)
        hs_ref[...] = h * dinv[:, None]

    return pl.pallas_call(
        body,
        grid=(n // r,),
        in_specs=[
            pl.BlockSpec((r, d), lambda i: (i, 0)),
            pl.BlockSpec((d, d), lambda i: (0, 0)),
            pl.BlockSpec((2, r, DEGW), lambda i: (0, i, 0)),
        ],
        out_specs=pl.BlockSpec((r, d), lambda i: (i, 0)),
        out_shape=jax.ShapeDtypeStruct((n, d), jnp.float32),
    )


def _tc_mid(n, d, r, relu):
    def body(aggp_ref, hsp_ref, b_ref, degp_ref, w_ref, out_ref):
        dinv = _dinv_from(degp_ref)
        t = (aggp_ref[0] + aggp_ref[1] + hsp_ref[...]) * dinv[:, None] + b_ref[...]
        if relu:
            t = jnp.maximum(t, 0.0)
        h = jnp.dot(t, w_ref[...], preferred_element_type=jnp.float32)
        out_ref[...] = h * dinv[:, None]

    return pl.pallas_call(
        body,
        grid=(n // r,),
        in_specs=[
            pl.BlockSpec((2, r, d), lambda i: (0, i, 0)),
            pl.BlockSpec((r, d), lambda i: (i, 0)),
            pl.BlockSpec((1, d), lambda i: (0, 0)),
            pl.BlockSpec((2, r, DEGW), lambda i: (0, i, 0)),
            pl.BlockSpec((d, d), lambda i: (0, 0)),
        ],
        out_specs=pl.BlockSpec((r, d), lambda i: (i, 0)),
        out_shape=jax.ShapeDtypeStruct((n, d), jnp.float32),
    )


def _tc_last(n, d, r):
    def body(aggp_ref, hsp_ref, degp_ref, out_ref):
        dinv = _dinv_from(degp_ref)
        out_ref[...] = (aggp_ref[0] + aggp_ref[1] + hsp_ref[...]) * dinv[:, None]

    return pl.pallas_call(
        body,
        grid=(n // r,),
        in_specs=[
            pl.BlockSpec((2, r, d), lambda i: (0, i, 0)),
            pl.BlockSpec((r, d), lambda i: (i, 0)),
            pl.BlockSpec((2, r, DEGW), lambda i: (0, i, 0)),
        ],
        out_specs=pl.BlockSpec((r, d), lambda i: (i, 0)),
        out_shape=jax.ShapeDtypeStruct((n, d), jnp.float32),
    )


def kernel(x, edge_index, W_in, b_in, W_h0, b_h0, W_h1, b_h1, W_out):
    n, d = x.shape
    e = edge_index.shape[1]
    r = 1000

    np_ = _pad_up(n + 1, NS * ZB)
    ep = _pad_up(e, NW * EK * 8)

    pad = ep - e
    srcp = jnp.concatenate(
        [edge_index[0], jnp.zeros((pad,), edge_index.dtype)]).reshape(ep // EK, EK)
    dstp = jnp.concatenate(
        [edge_index[1], jnp.full((pad,), n, edge_index.dtype)]).reshape(ep // EK, EK)

    degp = _sc_deg(np_, ep)(dstp)
    agg = _sc_agg(n, np_, ep, d)
    first = _tc_first(n, d, r)
    mid_nr = _tc_mid(n, d, r, relu=False)
    mid_re = _tc_mid(n, d, r, relu=True)
    last = _tc_last(n, d, r)

    b2_in = b_in.reshape(1, d)
    b2_h0 = b_h0.reshape(1, d)
    b2_h1 = b_h1.reshape(1, d)

    hs0 = first(x, W_in, degp)
    a0 = agg(hs0, srcp, dstp)
    hs1 = mid_nr(a0, hs0, b2_in, degp, W_h0)
    a1 = agg(hs1, srcp, dstp)
    hs2 = mid_re(a1, hs1, b2_h0, degp, W_h1)
    a2 = agg(hs2, srcp, dstp)
    hs3 = mid_re(a2, hs2, b2_h1, degp, W_out)
    a3 = agg(hs3, srcp, dstp)
    return last(a3, hs3, degp)
